# Initial kernel scaffold; baseline (speedup 1.0000x reference)
import jax
import jax.numpy as jnp
from jax.experimental import pallas as pl


def kernel(instructions, gamma, beta, idx_subject, idx_label):
    raise NotImplementedError("write your pallas kernel here")


# speedup vs baseline: 1.1370x; 1.1370x over previous
import functools

import jax
import jax.numpy as jnp
from jax import lax
from jax.experimental import pallas as pl
from jax.experimental.pallas import tpu as pltpu
from jax.experimental.pallas import tpu_sc as plsc

_NC = 2
_NS = 16
_NW = _NC * _NS
_LANES = 16

_EPS = 1e-5


def _rsqrt16(x):
    i = lax.bitcast_convert_type(x, jnp.int32)
    i = jnp.int32(0x5F3759DF) - lax.shift_right_logical(i, 1)
    y = lax.bitcast_convert_type(i, jnp.float32)
    half_x = 0.5 * x
    for _ in range(3):
        y = y * (1.5 - half_x * y * y)
    return y


def _lane_sum(x, perms):
    for perm in perms:
        x = x + x.at[perm].get(mode="promise_in_bounds")
    return x


def _make_norm_kernel(L, D, C, rows_per_chunk):
    n_chunks = L // rows_per_chunk
    chunks_per_w = -(-n_chunks // _NW)
    groups = D // C
    sub = C // _LANES

    mesh = plsc.VectorSubcoreMesh(core_axis_name="c", subcore_axis_name="s")

    @functools.partial(
        pl.kernel,
        mesh=mesh,
        out_type=jax.ShapeDtypeStruct((L, D), jnp.float32),
        scratch_types=[
            pltpu.VMEM((rows_per_chunk, D), jnp.float32),
            pltpu.VMEM((C,), jnp.float32),
            pltpu.VMEM((C,), jnp.float32),
        ],
    )
    def norm_kernel(tab_hbm, gamma_hbm, beta_hbm, out_hbm, buf_v, gam_v, bet_v):
        wid = lax.axis_index("s") * _NC + lax.axis_index("c")
        pltpu.sync_copy(gamma_hbm, gam_v)
        pltpu.sync_copy(beta_hbm, bet_v)
        lane = lax.iota(jnp.int32, _LANES)
        perms = tuple(lane ^ step for step in (8, 4, 2, 1))

        def normalize_row(row):
            def per_group(t, _):
                goff = t * C

                def acc(j, carry):
                    s, q = carry
                    v = row[pl.ds(goff + j * _LANES, _LANES)]
                    return s + v, q + v * v

                zeros = jnp.zeros((_LANES,), jnp.float32)
                s, q = lax.fori_loop(0, sub, acc, (zeros, zeros))
                inv_n = jnp.float32(1.0 / C)
                mean_v = _lane_sum(s, perms) * inv_n
                var_v = _lane_sum(q, perms) * inv_n - mean_v * mean_v
                rstd_v = _rsqrt16(var_v + _EPS)

                def norm(j, carry):
                    off = goff + j * _LANES
                    v = row[pl.ds(off, _LANES)]
                    g = gam_v[pl.ds(j * _LANES, _LANES)]
                    b = bet_v[pl.ds(j * _LANES, _LANES)]
                    row[pl.ds(off, _LANES)] = (v - mean_v) * rstd_v * g + b
                    return carry

                lax.fori_loop(0, sub, norm, 0)
                return _

            lax.fori_loop(0, groups, per_group, 0)

        for i in range(chunks_per_w):
            chunk = wid + _NW * i

            @pl.when(chunk < n_chunks)
            def _():
                base = chunk * rows_per_chunk
                pltpu.sync_copy(tab_hbm.at[pl.ds(base, rows_per_chunk)], buf_v)
                for r in range(rows_per_chunk):
                    normalize_row(buf_v.at[r])
                pltpu.sync_copy(buf_v, out_hbm.at[pl.ds(base, rows_per_chunk)])

    return norm_kernel


def _make_gather_kernel(L, D, B, rows_per_chunk):
    per_w = B // _NW
    n_chunks = per_w // rows_per_chunk

    mesh = plsc.VectorSubcoreMesh(core_axis_name="c", subcore_axis_name="s")

    @functools.partial(
        pl.kernel,
        mesh=mesh,
        out_type=jax.ShapeDtypeStruct((B, D), jnp.float32),
        scratch_types=[
            pltpu.VMEM((n_chunks, rows_per_chunk), jnp.int32),
            pltpu.VMEM((rows_per_chunk, D), jnp.float32),
            pltpu.VMEM((rows_per_chunk, D), jnp.float32),
            pltpu.SemaphoreType.DMA,
            pltpu.SemaphoreType.DMA,
            pltpu.SemaphoreType.DMA,
            pltpu.SemaphoreType.DMA,
        ],
    )
    def gather_kernel(tab_hbm, idx_hbm, out_hbm, idx_v, buf0, buf1, si0, si1, so0, so1):
        wid = lax.axis_index("s") * _NC + lax.axis_index("c")
        base = wid * per_w
        pltpu.sync_copy(idx_hbm.at[wid], idx_v)

        bufs = (buf0, buf1)
        sin = (si0, si1)
        sout = (so0, so1)
        in_h = [None] * n_chunks
        out_h = [None] * n_chunks
        in_h[0] = pltpu.async_copy(tab_hbm.at[idx_v.at[0]], bufs[0], sin[0])
        for c in range(n_chunks):
            p = c % 2
            in_h[c].wait()
            if c + 1 < n_chunks:
                if c >= 1:
                    out_h[c - 1].wait()
                in_h[c + 1] = pltpu.async_copy(
                    tab_hbm.at[idx_v.at[c + 1]], bufs[1 - p], sin[1 - p]
                )
            out_h[c] = pltpu.async_copy(
                bufs[p],
                out_hbm.at[pl.ds(base + c * rows_per_chunk, rows_per_chunk)],
                sout[p],
            )
        out_h[n_chunks - 2].wait()
        out_h[n_chunks - 1].wait()

    return gather_kernel


def kernel(instructions, gamma, beta, idx_subject, idx_label):
    S, L, T, C = instructions.shape
    B = idx_label.shape[0]
    D = T * C

    table = jnp.reshape(instructions[idx_subject], (L, D))

    rows_a = 8
    norm_fn = _make_norm_kernel(L, D, C, rows_a)
    norm_tab = norm_fn(table, gamma, beta)

    rows_b = 8
    per_w = B // _NW
    idx = jnp.reshape(idx_label.astype(jnp.int32), (_NW, per_w // rows_b, rows_b))
    gather_fn = _make_gather_kernel(L, D, B, rows_b)
    out = gather_fn(norm_tab, idx)

    return jnp.reshape(out, (B, T, C))
